# baseline (speedup 1.0000x reference)
import jax
import jax.numpy as jnp
from jax import lax
from jax.experimental import pallas as pl
from jax.experimental.pallas import tpu as pltpu


def _l1_kernel(xa_ref, xb_ref, w_ref, scale_ref, shift_ref, o_ref):
    _, n, wp2 = xa_ref.shape
    W = wp2 - 2
    R = n // 2
    Wp = W // 2
    cout = o_ref.shape[1]

    x = jnp.concatenate([xa_ref[0], xb_ref[0, :2]], axis=0)
    taps = [x[ky:ky + n, kx:kx + W] for ky in range(3) for kx in range(3)]

    cms = []
    for co in range(cout):
        acc = taps[0] * w_ref[co]
        for t in range(1, 9):
            acc = acc + taps[t] * w_ref[t * cout + co]
        y = jnp.maximum(acc * scale_ref[co] + shift_ref[co], 0.0)
        rm = jnp.max(y.reshape(R, 2, W), axis=1)
        cm = jnp.maximum(rm, jnp.concatenate([rm[:, 1:], rm[:, :1]], axis=1))
        cms.append(cm)
    cm_all = jnp.stack(cms, axis=0).reshape(cout * R, W)

    wi = lax.broadcasted_iota(jnp.int32, (W, Wp), 0)
    pi = lax.broadcasted_iota(jnp.int32, (W, Wp), 1)
    sel = (wi == 2 * pi).astype(jnp.float32)
    pooled = jnp.dot(cm_all, sel, preferred_element_type=jnp.float32,
                     precision=lax.Precision.HIGHEST)
    o_ref[0] = pooled.reshape(cout, R, Wp)


def _l1_call(x_img, w, scale, shift, *, rows_per_step=56):
    B, H, W = x_img.shape
    Cout = w.shape[-1]
    R = rows_per_step
    n = 2 * R
    Hp, Wp = H // 2, W // 2
    assert Hp % R == 0 and n % 8 == 0
    xpad = jnp.pad(x_img, ((0, 0), (1, 7), (1, 1)))
    w_flat = w.reshape(-1)
    return pl.pallas_call(
        _l1_kernel,
        out_shape=jax.ShapeDtypeStruct((B, Cout, Hp, Wp), jnp.float32),
        grid_spec=pltpu.PrefetchScalarGridSpec(
            num_scalar_prefetch=0,
            grid=(B, Hp // R),
            in_specs=[
                pl.BlockSpec((1, n, W + 2), lambda b, r: (b, r, 0)),
                pl.BlockSpec((1, 8, W + 2),
                             lambda b, r: (b, (r + 1) * (n // 8), 0)),
                pl.BlockSpec(memory_space=pltpu.MemorySpace.SMEM),
                pl.BlockSpec(memory_space=pltpu.MemorySpace.SMEM),
                pl.BlockSpec(memory_space=pltpu.MemorySpace.SMEM),
            ],
            out_specs=pl.BlockSpec((1, Cout, R, Wp), lambda b, r: (b, 0, r, 0)),
        ),
        compiler_params=pltpu.CompilerParams(
            dimension_semantics=("parallel", "parallel"),
            vmem_limit_bytes=64 * 1024 * 1024),
    )(xpad, xpad, w_flat, scale.reshape(-1), shift.reshape(-1))


def _conv_kernel(xa_ref, xb_ref, w_ref, scale_ref, shift_ref, o_ref):
    _, n, wp2, cin = xa_ref.shape
    R = n // 2
    W = wp2 - 2
    Wp = W // 2
    cout = o_ref.shape[-1]

    x_all = jnp.concatenate([xa_ref[0], xb_ref[0]], axis=0)
    xcat = jnp.concatenate([x_all[:, kx:kx + W, :] for kx in range(3)],
                           axis=-1)
    acc = jnp.dot(xcat[0:n].reshape(n * W, 3 * cin), w_ref[0],
                  preferred_element_type=jnp.float32)
    acc += jnp.dot(xcat[1:1 + n].reshape(n * W, 3 * cin), w_ref[1],
                   preferred_element_type=jnp.float32)
    acc += jnp.dot(xcat[2:2 + n].reshape(n * W, 3 * cin), w_ref[2],
                   preferred_element_type=jnp.float32)

    y = jnp.maximum(acc * scale_ref[...] + shift_ref[...], 0.0)
    rm = jnp.max(y.reshape(R, 2, W, cout), axis=1)
    pooled = jnp.max(rm.reshape(R, Wp, 2, cout), axis=2)
    o_ref[0] = pooled.astype(o_ref.dtype)


def _conv_call(x, w3d, scale, shift, *, rows_per_step):
    B, H, W, Cin = x.shape
    Cout = w3d.shape[-1]
    Hp, Wp = H // 2, W // 2
    R = rows_per_step
    assert Hp % R == 0
    xp = jnp.pad(x, ((0, 0), (1, 1), (1, 1), (0, 0)))
    return pl.pallas_call(
        _conv_kernel,
        out_shape=jax.ShapeDtypeStruct((B, Hp, Wp, Cout), jnp.float32),
        grid_spec=pltpu.PrefetchScalarGridSpec(
            num_scalar_prefetch=0,
            grid=(B, Hp // R),
            in_specs=[
                pl.BlockSpec((1, 2 * R, W + 2, Cin), lambda b, r: (b, r, 0, 0)),
                pl.BlockSpec((1, 2, W + 2, Cin), lambda b, r: (b, R * (r + 1), 0, 0)),
                pl.BlockSpec((3, 3 * Cin, Cout), lambda b, r: (0, 0, 0)),
                pl.BlockSpec((1, Cout), lambda b, r: (0, 0)),
                pl.BlockSpec((1, Cout), lambda b, r: (0, 0)),
            ],
            out_specs=pl.BlockSpec((1, R, Wp, Cout), lambda b, r: (b, r, 0, 0)),
        ),
        compiler_params=pltpu.CompilerParams(
            dimension_semantics=("parallel", "parallel"),
            vmem_limit_bytes=64 * 1024 * 1024),
    )(xp, xp, w3d, scale, shift)


def _fc_kernel(x_ref, w1_ref, b1_ref, w2_ref, o_ref, acc_ref):
    k = pl.program_id(1)

    @pl.when(k == 0)
    def _():
        acc_ref[...] = jnp.zeros_like(acc_ref)

    acc_ref[...] += jnp.dot(x_ref[...], w1_ref[...],
                            preferred_element_type=jnp.float32)

    @pl.when(k == pl.num_programs(1) - 1)
    def _():
        h = jnp.maximum(acc_ref[...] + b1_ref[...], 0.0)
        o_ref[0] = jnp.dot(h, w2_ref[...],
                           preferred_element_type=jnp.float32).astype(o_ref.dtype)


def _fc_call(x, w1, b1, w2, b2, *, tk=1792, col_tiles=2):
    B, K = x.shape
    N1 = w1.shape[1]
    N2 = w2.shape[1]
    assert K % tk == 0 and N1 % col_tiles == 0
    nk = K // tk
    nh = N1 // col_tiles
    partials = pl.pallas_call(
        _fc_kernel,
        out_shape=jax.ShapeDtypeStruct((col_tiles, B, N2), jnp.float32),
        grid_spec=pltpu.PrefetchScalarGridSpec(
            num_scalar_prefetch=0,
            grid=(col_tiles, nk),
            in_specs=[
                pl.BlockSpec((B, tk), lambda j, k: (0, k)),
                pl.BlockSpec((tk, nh), lambda j, k: (k, j)),
                pl.BlockSpec((1, nh), lambda j, k: (0, j)),
                pl.BlockSpec((nh, N2), lambda j, k: (j, 0)),
            ],
            out_specs=pl.BlockSpec((1, B, N2), lambda j, k: (j, 0, 0)),
            scratch_shapes=[pltpu.VMEM((B, nh), jnp.float32)],
        ),
        compiler_params=pltpu.CompilerParams(
            dimension_semantics=("parallel", "arbitrary"),
            vmem_limit_bytes=64 * 1024 * 1024),
    )(x, w1, b1, w2)
    return jnp.sum(partials, axis=0) + b2


@jax.jit
def _forward(x_nchw,
             conv0_w, conv0_scale, conv0_shift,
             conv1_w, conv1_scale, conv1_shift,
             conv2_w, conv2_scale, conv2_shift,
             conv3_w, conv3_scale, conv3_shift,
             conv4_w, conv4_scale, conv4_shift,
             w1, b1, w2, b2):
    B = x_nchw.shape[0]

    y1 = _l1_call(x_nchw[:, 0], conv0_w, conv0_scale, conv0_shift,
                  rows_per_step=56)
    x = jnp.transpose(y1, (0, 2, 3, 1))

    conv_rest = ((conv1_w, conv1_scale, conv1_shift, 28),
                 (conv2_w, conv2_scale, conv2_shift, 14),
                 (conv3_w, conv3_scale, conv3_shift, 14),
                 (conv4_w, conv4_scale, conv4_shift, 7))
    for w, scale, shift, R in conv_rest:
        cin, cout = w.shape[2], w.shape[3]
        w3d = w.reshape(3, 3 * cin, cout)
        x = _conv_call(x, w3d, scale, shift, rows_per_step=R)

    feats = jnp.transpose(x, (0, 3, 1, 2)).reshape(B, -1)
    return _fc_call(feats, w1, b1, w2, b2)


def kernel(x_nchw, conv0_w, conv0_scale, conv0_shift, conv1_w, conv1_scale,
           conv1_shift, conv2_w, conv2_scale, conv2_shift, conv3_w,
           conv3_scale, conv3_shift, conv4_w, conv4_scale, conv4_shift,
           w1, b1, w2, b2):
    return _forward(x_nchw,
                    conv0_w, conv0_scale, conv0_shift,
                    conv1_w, conv1_scale, conv1_shift,
                    conv2_w, conv2_scale, conv2_shift,
                    conv3_w, conv3_scale, conv3_shift,
                    conv4_w, conv4_scale, conv4_shift,
                    w1, b1, w2, b2)


# speedup vs baseline: 1.1448x; 1.1448x over previous
import jax
import jax.numpy as jnp
from jax import lax
from jax.experimental import pallas as pl
from jax.experimental.pallas import tpu as pltpu


def _l1_kernel(xa_ref, xb_ref, w_ref, scale_ref, shift_ref, o_ref):
    _, n, wp2 = xa_ref.shape
    W = wp2 - 2
    R = n // 2
    Wp = W // 2
    cout = o_ref.shape[1]

    x = jnp.concatenate([xa_ref[0], xb_ref[0, :2]], axis=0)
    taps = [x[ky:ky + n, kx:kx + W] for ky in range(3) for kx in range(3)]

    cms = []
    for co in range(cout):
        acc = taps[0] * w_ref[co]
        for t in range(1, 9):
            acc = acc + taps[t] * w_ref[t * cout + co]
        y = jnp.maximum(acc * scale_ref[co] + shift_ref[co], 0.0)
        rm = jnp.max(y.reshape(R, 2, W), axis=1)
        cm = jnp.maximum(rm, jnp.concatenate([rm[:, 1:], rm[:, :1]], axis=1))
        cms.append(cm)
    cm_all = jnp.stack(cms, axis=0).reshape(cout * R, W)

    wi = lax.broadcasted_iota(jnp.int32, (W, Wp), 0)
    pi = lax.broadcasted_iota(jnp.int32, (W, Wp), 1)
    sel = (wi == 2 * pi).astype(jnp.float32)
    pooled = jnp.dot(cm_all, sel, preferred_element_type=jnp.float32,
                     precision=lax.Precision.HIGHEST)
    o_ref[0] = pooled.reshape(cout, R, Wp).astype(o_ref.dtype)


def _l1_call(x_img, w, scale, shift, *, rows_per_step=56):
    B, H, W = x_img.shape
    Cout = w.shape[-1]
    R = rows_per_step
    n = 2 * R
    Hp, Wp = H // 2, W // 2
    assert Hp % R == 0 and n % 8 == 0
    xpad = jnp.pad(x_img, ((0, 0), (1, 7), (1, 1)))
    w_flat = w.reshape(-1)
    return pl.pallas_call(
        _l1_kernel,
        out_shape=jax.ShapeDtypeStruct((B, Cout, Hp, Wp), jnp.bfloat16),
        grid_spec=pltpu.PrefetchScalarGridSpec(
            num_scalar_prefetch=0,
            grid=(B, Hp // R),
            in_specs=[
                pl.BlockSpec((1, n, W + 2), lambda b, r: (b, r, 0)),
                pl.BlockSpec((1, 8, W + 2),
                             lambda b, r: (b, (r + 1) * (n // 8), 0)),
                pl.BlockSpec(memory_space=pltpu.MemorySpace.SMEM),
                pl.BlockSpec(memory_space=pltpu.MemorySpace.SMEM),
                pl.BlockSpec(memory_space=pltpu.MemorySpace.SMEM),
            ],
            out_specs=pl.BlockSpec((1, Cout, R, Wp), lambda b, r: (b, 0, r, 0)),
        ),
        compiler_params=pltpu.CompilerParams(
            dimension_semantics=("parallel", "parallel"),
            vmem_limit_bytes=64 * 1024 * 1024),
    )(xpad, xpad, w_flat, scale.reshape(-1), shift.reshape(-1))


def _conv_kernel(xa_ref, xb_ref, w_ref, scale_ref, shift_ref, o_ref):
    _, n, wp2, cin = xa_ref.shape
    R = n // 2
    W = wp2 - 2
    Wp = W // 2
    cout = o_ref.shape[-1]

    x_all = jnp.concatenate([xa_ref[0], xb_ref[0]], axis=0)
    xcat = jnp.concatenate([x_all[:, kx:kx + W, :] for kx in range(3)],
                           axis=-1)
    acc = jnp.dot(xcat[0:n].reshape(n * W, 3 * cin), w_ref[0],
                  preferred_element_type=jnp.float32)
    acc += jnp.dot(xcat[1:1 + n].reshape(n * W, 3 * cin), w_ref[1],
                   preferred_element_type=jnp.float32)
    acc += jnp.dot(xcat[2:2 + n].reshape(n * W, 3 * cin), w_ref[2],
                   preferred_element_type=jnp.float32)

    y = jnp.maximum(acc * scale_ref[...] + shift_ref[...], 0.0)
    rm = jnp.max(y.reshape(R, 2, W, cout), axis=1)
    pooled = jnp.max(rm.reshape(R, Wp, 2, cout), axis=2)
    o_ref[0] = pooled.astype(o_ref.dtype)


def _conv_call(x, w3d, scale, shift, *, rows_per_step):
    B, H, W, Cin = x.shape
    Cout = w3d.shape[-1]
    Hp, Wp = H // 2, W // 2
    R = rows_per_step
    assert Hp % R == 0
    xp = jnp.pad(x, ((0, 0), (1, 1), (1, 1), (0, 0)))
    return pl.pallas_call(
        _conv_kernel,
        out_shape=jax.ShapeDtypeStruct((B, Hp, Wp, Cout), jnp.bfloat16),
        grid_spec=pltpu.PrefetchScalarGridSpec(
            num_scalar_prefetch=0,
            grid=(B, Hp // R),
            in_specs=[
                pl.BlockSpec((1, 2 * R, W + 2, Cin), lambda b, r: (b, r, 0, 0)),
                pl.BlockSpec((1, 2, W + 2, Cin), lambda b, r: (b, R * (r + 1), 0, 0)),
                pl.BlockSpec((3, 3 * Cin, Cout), lambda b, r: (0, 0, 0)),
                pl.BlockSpec((1, Cout), lambda b, r: (0, 0)),
                pl.BlockSpec((1, Cout), lambda b, r: (0, 0)),
            ],
            out_specs=pl.BlockSpec((1, R, Wp, Cout), lambda b, r: (b, r, 0, 0)),
        ),
        compiler_params=pltpu.CompilerParams(
            dimension_semantics=("parallel", "parallel"),
            vmem_limit_bytes=64 * 1024 * 1024),
    )(xp, xp, w3d, scale, shift)


def _fc_kernel(x_ref, w1_ref, b1_ref, w2_ref, o_ref, acc_ref):
    k = pl.program_id(1)

    @pl.when(k == 0)
    def _():
        acc_ref[...] = jnp.zeros_like(acc_ref)

    acc_ref[...] += jnp.dot(x_ref[...], w1_ref[...],
                            preferred_element_type=jnp.float32)

    @pl.when(k == pl.num_programs(1) - 1)
    def _():
        h = jnp.maximum(acc_ref[...] + b1_ref[...], 0.0)
        o_ref[0] = jnp.dot(h.astype(jnp.bfloat16), w2_ref[...],
                           preferred_element_type=jnp.float32).astype(o_ref.dtype)


def _fc_call(x, w1, b1, w2, b2, *, tk=1792, col_tiles=2):
    B, K = x.shape
    N1 = w1.shape[1]
    N2 = w2.shape[1]
    assert K % tk == 0 and N1 % col_tiles == 0
    nk = K // tk
    nh = N1 // col_tiles
    partials = pl.pallas_call(
        _fc_kernel,
        out_shape=jax.ShapeDtypeStruct((col_tiles, B, N2), jnp.float32),
        grid_spec=pltpu.PrefetchScalarGridSpec(
            num_scalar_prefetch=0,
            grid=(col_tiles, nk),
            in_specs=[
                pl.BlockSpec((B, tk), lambda j, k: (0, k)),
                pl.BlockSpec((tk, nh), lambda j, k: (k, j)),
                pl.BlockSpec((1, nh), lambda j, k: (0, j)),
                pl.BlockSpec((nh, N2), lambda j, k: (j, 0)),
            ],
            out_specs=pl.BlockSpec((1, B, N2), lambda j, k: (j, 0, 0)),
            scratch_shapes=[pltpu.VMEM((B, nh), jnp.float32)],
        ),
        compiler_params=pltpu.CompilerParams(
            dimension_semantics=("parallel", "arbitrary"),
            vmem_limit_bytes=64 * 1024 * 1024),
    )(x, w1, b1, w2)
    return jnp.sum(partials, axis=0) + b2


@jax.jit
def _forward(x_nchw,
             conv0_w, conv0_scale, conv0_shift,
             conv1_w, conv1_scale, conv1_shift,
             conv2_w, conv2_scale, conv2_shift,
             conv3_w, conv3_scale, conv3_shift,
             conv4_w, conv4_scale, conv4_shift,
             w1, b1, w2, b2):
    B = x_nchw.shape[0]

    y1 = _l1_call(x_nchw[:, 0], conv0_w, conv0_scale, conv0_shift,
                  rows_per_step=56)
    x = jnp.transpose(y1, (0, 2, 3, 1))

    conv_rest = ((conv1_w, conv1_scale, conv1_shift, 28),
                 (conv2_w, conv2_scale, conv2_shift, 14),
                 (conv3_w, conv3_scale, conv3_shift, 14),
                 (conv4_w, conv4_scale, conv4_shift, 7))
    for w, scale, shift, R in conv_rest:
        cin, cout = w.shape[2], w.shape[3]
        w3d = w.reshape(3, 3 * cin, cout).astype(jnp.bfloat16)
        x = _conv_call(x, w3d, scale, shift, rows_per_step=R)

    feats = jnp.transpose(x, (0, 3, 1, 2)).reshape(B, -1)
    return _fc_call(feats.astype(jnp.bfloat16), w1.astype(jnp.bfloat16),
                    b1, w2.astype(jnp.bfloat16), b2)


def kernel(x_nchw, conv0_w, conv0_scale, conv0_shift, conv1_w, conv1_scale,
           conv1_shift, conv2_w, conv2_scale, conv2_shift, conv3_w,
           conv3_scale, conv3_shift, conv4_w, conv4_scale, conv4_shift,
           w1, b1, w2, b2):
    return _forward(x_nchw,
                    conv0_w, conv0_scale, conv0_shift,
                    conv1_w, conv1_scale, conv1_shift,
                    conv2_w, conv2_scale, conv2_shift,
                    conv3_w, conv3_scale, conv3_shift,
                    conv4_w, conv4_scale, conv4_shift,
                    w1, b1, w2, b2)
